# baseline (speedup 1.0000x reference)
import functools

import jax
import jax.numpy as jnp
from jax import lax
from jax.experimental import pallas as pl
from jax.experimental.pallas import tpu as pltpu
from jax.experimental.pallas import tpu_sc as plsc

EMBED_DIM = 64
VALUES_LEN = 81920
NUM_TABLES = 4
NUM_WORKERS = 32
WORKERS_PER_TABLE = NUM_WORKERS // NUM_TABLES
PER_W = VALUES_LEN // WORKERS_PER_TABLE
CHUNK = 128
NCHUNK = PER_W // CHUNK

_mesh = plsc.VectorSubcoreMesh(core_axis_name="c", subcore_axis_name="s")


@functools.partial(
    pl.kernel,
    mesh=_mesh,
    out_type=jax.ShapeDtypeStruct((NUM_TABLES * VALUES_LEN, EMBED_DIM),
                                  jnp.float32),
    scratch_types=[
        pltpu.VMEM((NCHUNK, CHUNK), jnp.int32),
        pltpu.VMEM((CHUNK, EMBED_DIM), jnp.float32),
        pltpu.SemaphoreType.DMA,
    ],
    compiler_params=pltpu.CompilerParams(use_tc_tiling_on_sc=False),
)
def _grouped_embedding(v0, v1, v2, v3, w0, w1, w2, w3, out,
                       idx_v, rows_v, sem):
    wid = lax.axis_index("s") * 2 + lax.axis_index("c")
    table_id = wid // WORKERS_PER_TABLE
    sub = wid % WORKERS_PER_TABLE

    for t, (vals, table) in enumerate(
        ((v0, w0), (v1, w1), (v2, w2), (v3, w3))):

        @pl.when(table_id == t)
        def _(vals=vals, table=table, t=t):
            row_base = sub * NCHUNK
            out_base = t * VALUES_LEN + sub * PER_W
            pltpu.sync_copy(vals.at[pl.ds(row_base, NCHUNK), :], idx_v)

            def body(j, carry):
                pltpu.async_copy(table.at[idx_v.at[j]], rows_v, sem).wait()
                pltpu.sync_copy(
                    rows_v, out.at[pl.ds(out_base + j * CHUNK, CHUNK), :])
                return carry

            lax.fori_loop(0, NCHUNK, body, 0)


def kernel(values_0, values_1, values_2, values_3, W0, W1, W2, W3):
    v0 = values_0.reshape(VALUES_LEN // CHUNK, CHUNK)
    v1 = values_1.reshape(VALUES_LEN // CHUNK, CHUNK)
    v2 = values_2.reshape(VALUES_LEN // CHUNK, CHUNK)
    v3 = values_3.reshape(VALUES_LEN // CHUNK, CHUNK)
    return _grouped_embedding(v0, v1, v2, v3, W0, W1, W2, W3)


# speedup vs baseline: 1.0376x; 1.0376x over previous
import functools

import jax
import jax.numpy as jnp
from jax import lax
from jax.experimental import pallas as pl
from jax.experimental.pallas import tpu as pltpu
from jax.experimental.pallas import tpu_sc as plsc

EMBED_DIM = 64
VALUES_LEN = 81920
NUM_TABLES = 4
NUM_WORKERS = 32
WORKERS_PER_TABLE = NUM_WORKERS // NUM_TABLES
PER_W = VALUES_LEN // WORKERS_PER_TABLE
CHUNK = 128
NCHUNK = PER_W // CHUNK
NBUF = 8

_mesh = plsc.VectorSubcoreMesh(core_axis_name="c", subcore_axis_name="s")


@functools.partial(
    pl.kernel,
    mesh=_mesh,
    out_type=jax.ShapeDtypeStruct((NUM_TABLES * VALUES_LEN, EMBED_DIM),
                                  jnp.float32),
    scratch_types=[
        pltpu.VMEM((NCHUNK, CHUNK), jnp.int32),
        pltpu.VMEM((NBUF, CHUNK, EMBED_DIM), jnp.float32),
        pltpu.SemaphoreType.DMA((NBUF,)),
        pltpu.SemaphoreType.DMA((NBUF,)),
    ],
    compiler_params=pltpu.CompilerParams(use_tc_tiling_on_sc=False),
)
def _grouped_embedding(v0, v1, v2, v3, w0, w1, w2, w3, out,
                       idx_v, rows_v, sem_g, sem_w):
    wid = lax.axis_index("s") * 2 + lax.axis_index("c")
    table_id = wid // WORKERS_PER_TABLE
    sub = wid % WORKERS_PER_TABLE

    for t, (vals, table) in enumerate(
        ((v0, w0), (v1, w1), (v2, w2), (v3, w3))):

        @pl.when(table_id == t)
        def _(vals=vals, table=table, t=t):
            row_base = sub * NCHUNK
            out_base = t * VALUES_LEN + sub * PER_W
            pltpu.sync_copy(vals.at[pl.ds(row_base, NCHUNK), :], idx_v)

            def out_slice(c):
                return out.at[pl.ds(out_base + c * CHUNK, CHUNK), :]

            def body(i, carry):
                gathers = []
                for b in range(NBUF):
                    c = i * NBUF + b

                    @pl.when(i > 0)
                    def _(b=b, c=c):
                        pltpu.make_async_copy(
                            rows_v.at[b], out_slice(c), sem_w.at[b]).wait()

                    gathers.append(pltpu.async_copy(
                        table.at[idx_v.at[c]], rows_v.at[b], sem_g.at[b]))
                for b in range(NBUF):
                    c = i * NBUF + b
                    gathers[b].wait()
                    pltpu.async_copy(rows_v.at[b], out_slice(c), sem_w.at[b])
                return carry

            lax.fori_loop(0, NCHUNK // NBUF, body, 0)
            for b in range(NBUF):
                pltpu.make_async_copy(
                    rows_v.at[b], out_slice(b), sem_w.at[b]).wait()


def kernel(values_0, values_1, values_2, values_3, W0, W1, W2, W3):
    v0 = values_0.reshape(VALUES_LEN // CHUNK, CHUNK)
    v1 = values_1.reshape(VALUES_LEN // CHUNK, CHUNK)
    v2 = values_2.reshape(VALUES_LEN // CHUNK, CHUNK)
    v3 = values_3.reshape(VALUES_LEN // CHUNK, CHUNK)
    return _grouped_embedding(v0, v1, v2, v3, W0, W1, W2, W3)


# speedup vs baseline: 1.0388x; 1.0011x over previous
import functools

import jax
import jax.numpy as jnp
from jax import lax
from jax.experimental import pallas as pl
from jax.experimental.pallas import tpu as pltpu
from jax.experimental.pallas import tpu_sc as plsc

EMBED_DIM = 64
VALUES_LEN = 81920
NUM_TABLES = 4
NUM_WORKERS = 32
WORKERS_PER_TABLE = NUM_WORKERS // NUM_TABLES
PER_W = VALUES_LEN // WORKERS_PER_TABLE
CHUNK = 128
NCHUNK = PER_W // CHUNK
NBUF = 8

_mesh = plsc.VectorSubcoreMesh(core_axis_name="c", subcore_axis_name="s")


@functools.partial(
    pl.kernel,
    mesh=_mesh,
    out_type=jax.ShapeDtypeStruct((NUM_TABLES * VALUES_LEN, EMBED_DIM),
                                  jnp.float32),
    scratch_types=[
        pltpu.VMEM((PER_W,), jnp.int32),
        pltpu.VMEM((NBUF, CHUNK, EMBED_DIM), jnp.float32),
        pltpu.SemaphoreType.DMA((NBUF,)),
        pltpu.SemaphoreType.DMA((NBUF,)),
    ],
    compiler_params=pltpu.CompilerParams(use_tc_tiling_on_sc=False),
)
def _grouped_embedding(v0, v1, v2, v3, w0, w1, w2, w3, out,
                       idx_v, rows_v, sem_g, sem_w):
    wid = lax.axis_index("s") * 2 + lax.axis_index("c")
    table_id = wid // WORKERS_PER_TABLE
    sub = wid % WORKERS_PER_TABLE

    for t, (vals, table) in enumerate(
        ((v0, w0), (v1, w1), (v2, w2), (v3, w3))):

        @pl.when(table_id == t)
        def _(vals=vals, table=table, t=t):
            out_base = t * VALUES_LEN + sub * PER_W
            pltpu.sync_copy(vals.at[pl.ds(sub * PER_W, PER_W)], idx_v)

            def out_slice(c):
                return out.at[pl.ds(out_base + c * CHUNK, CHUNK), :]

            def body(i, carry):
                gathers = []
                for b in range(NBUF):
                    c = i * NBUF + b

                    @pl.when(i > 0)
                    def _(b=b, c=c):
                        pltpu.make_async_copy(
                            rows_v.at[b], out_slice(c), sem_w.at[b]).wait()

                    gathers.append(pltpu.async_copy(
                        table.at[idx_v.at[pl.ds(c * CHUNK, CHUNK)]],
                        rows_v.at[b], sem_g.at[b]))
                for b in range(NBUF):
                    c = i * NBUF + b
                    gathers[b].wait()
                    pltpu.async_copy(rows_v.at[b], out_slice(c), sem_w.at[b])
                return carry

            lax.fori_loop(0, NCHUNK // NBUF, body, 0)
            for b in range(NBUF):
                pltpu.make_async_copy(
                    rows_v.at[b], out_slice(b), sem_w.at[b]).wait()


def kernel(values_0, values_1, values_2, values_3, W0, W1, W2, W3):
    return _grouped_embedding(values_0, values_1, values_2, values_3,
                              W0, W1, W2, W3)
